# Initial kernel scaffold; baseline (speedup 1.0000x reference)
import jax
import jax.numpy as jnp
from jax.experimental import pallas as pl


def kernel(x, edge_index, W_self1, W_neigh1, b1, W_self2, W_neigh2, b2, W_mu, b_mu, W_var, b_var, eps):
    raise NotImplementedError("write your pallas kernel here")


# speedup vs baseline: 3.0114x; 3.0114x over previous
import functools

import jax
import jax.numpy as jnp
from jax import lax
from jax.experimental import pallas as pl
from jax.experimental.pallas import tpu as pltpu
from jax.experimental.pallas import tpu_sc as plsc

_N = 10000
_E = 320000
_D = 128
_NT = 32
_CH = 128
_GB = 8
_K = _GB * (-(-_E // (_NT * _CH * _GB)))
_KG = _K // _GB
_EP = _NT * _K * _CH
_NP = 10240
_RPT = _NP // 16
_NS = _RPT // _CH
_HR = _NP // _D
_L = 16

_f32 = jnp.float32


def _agg_body(h_hbm, srcm, dstm, z128, acc_out,
              src_v, dst_v, rows_v, acc_sh, sem):
    c = lax.axis_index("c")
    s = lax.axis_index("s")
    t = s * 2 + c
    base = s * _RPT

    pltpu.sync_copy(z128, rows_v)
    for i in range(_NS):
        pltpu.sync_copy(rows_v, acc_sh.at[pl.ds(base + i * _CH, _CH)])
    plsc.subcore_barrier()

    @pl.loop(0, _KG)
    def _group(g):
        pltpu.sync_copy(srcm.at[t, pl.ds(g * _GB, _GB)], src_v)
        pltpu.sync_copy(dstm.at[t, pl.ds(g * _GB, _GB)], dst_v)
        for j in range(_GB):
            pltpu.async_copy(h_hbm.at[src_v.at[j]], rows_v, sem).wait()
            pltpu.sync_copy(rows_v, acc_sh.at[dst_v.at[j]], add=True)

    plsc.subcore_barrier()
    for i in range(_NS):
        pltpu.sync_copy(acc_sh.at[pl.ds(base + i * _CH, _CH)], rows_v)
        pltpu.sync_copy(rows_v, acc_out.at[c, pl.ds(base + i * _CH, _CH)])


def _deg_body(dstm, z128, ones128, deg_out, dst_v, ones_v, acc_sh):
    c = lax.axis_index("c")
    s = lax.axis_index("s")
    t = s * 2 + c
    base = s * _RPT

    pltpu.sync_copy(z128, ones_v)
    for i in range(_NS):
        pltpu.sync_copy(ones_v, acc_sh.at[pl.ds(base + i * _CH, _CH)])
    pltpu.sync_copy(ones128, ones_v)
    plsc.subcore_barrier()

    @pl.loop(0, _KG)
    def _group(g):
        pltpu.sync_copy(dstm.at[t, pl.ds(g * _GB, _GB)], dst_v)
        for j in range(_GB):
            pltpu.sync_copy(ones_v, acc_sh.at[dst_v.at[j]], add=True)

    plsc.subcore_barrier()
    for i in range(_NS):
        pltpu.sync_copy(acc_sh.at[pl.ds(base + i * _CH, _CH)], ones_v)
        pltpu.sync_copy(ones_v, deg_out.at[c, pl.ds(base + i * _CH, _CH)])


_sc_mesh = plsc.VectorSubcoreMesh(core_axis_name="c", subcore_axis_name="s")

_agg = pl.kernel(
    _agg_body,
    out_type=jax.ShapeDtypeStruct((2, _NP, _D), _f32),
    mesh=_sc_mesh,
    scratch_types=[
        pltpu.VMEM((_GB, _CH), jnp.int32),
        pltpu.VMEM((_GB, _CH), jnp.int32),
        pltpu.VMEM((_CH, _D), _f32),
        pltpu.VMEM_SHARED((_NP, _D), _f32),
        pltpu.SemaphoreType.DMA,
    ],
)

_deg_count = pl.kernel(
    _deg_body,
    out_type=jax.ShapeDtypeStruct((2, _NP, _D), _f32),
    mesh=_sc_mesh,
    scratch_types=[
        pltpu.VMEM((_GB, _CH), jnp.int32),
        pltpu.VMEM((_CH, _D), _f32),
        pltpu.VMEM_SHARED((_NP, _D), _f32),
    ],
)


_BN = 2048
_BH = _BN // _D


def _log1p_body(x_ref, o_ref):
    o_ref[...] = jnp.log(x_ref[...] + 1.0)


def _neigh_mean(acc_ref, deg_ref):
    a = acc_ref[0] + acc_ref[1]
    deg = deg_ref[0, :, 0:1] + deg_ref[1, :, 0:1]
    return a / jnp.maximum(deg, 1.0)


def _dense1_body(h_ref, acc_ref, deg_ref, ws_ref, wn_ref, b_ref, o_ref):
    hn = _neigh_mean(acc_ref, deg_ref)
    z = (jnp.dot(h_ref[...], ws_ref[...], preferred_element_type=_f32)
         + jnp.dot(hn, wn_ref[...], preferred_element_type=_f32)
         + b_ref[...])
    z = jnp.maximum(z, 0.0)
    nrm = jnp.sqrt(jnp.sum(z * z, axis=1, keepdims=True))
    o_ref[...] = z / jnp.maximum(nrm, 1e-12)


def _dense2_body(h_ref, acc_ref, deg_ref, ws_ref, wn_ref, b_ref,
                 wmu_ref, bmu_ref, wvar_ref, bvar_ref, eps_ref,
                 out_ref, mu_ref, var_ref):
    hn = _neigh_mean(acc_ref, deg_ref)
    h2 = (jnp.dot(h_ref[...], ws_ref[...], preferred_element_type=_f32)
          + jnp.dot(hn, wn_ref[...], preferred_element_type=_f32)
          + b_ref[...])
    mu = jnp.dot(h2, wmu_ref[...], preferred_element_type=_f32) + bmu_ref[...]
    var = jnp.dot(h2, wvar_ref[...], preferred_element_type=_f32) + bvar_ref[...]
    mu_ref[...] = mu
    var_ref[...] = var
    out_ref[...] = mu + jnp.sqrt(jnp.exp(var) + 1e-8) * eps_ref[...]


_row_spec = pl.BlockSpec((_BN, _D), lambda i: (i, 0))
_acc_spec = pl.BlockSpec((2, _BN, _D), lambda i: (0, i, 0))
_deg_spec = pl.BlockSpec((2, _BN, _D), lambda i: (0, i, 0))
_w_spec = pl.BlockSpec((_D, _D), lambda i: (0, 0))
_b_spec = pl.BlockSpec((1, _D), lambda i: (0, 0))
_GRID = (-(-_N // _BN),)

_log1p = pl.pallas_call(
    _log1p_body,
    grid=_GRID,
    in_specs=[_row_spec],
    out_specs=_row_spec,
    out_shape=jax.ShapeDtypeStruct((_N, _D), _f32),
)

_dense1 = pl.pallas_call(
    _dense1_body,
    grid=_GRID,
    in_specs=[_row_spec, _acc_spec, _deg_spec, _w_spec, _w_spec, _b_spec],
    out_specs=_row_spec,
    out_shape=jax.ShapeDtypeStruct((_N, _D), _f32),
)

_dense2 = pl.pallas_call(
    _dense2_body,
    grid=_GRID,
    in_specs=[_row_spec, _acc_spec, _deg_spec, _w_spec, _w_spec, _b_spec,
              _w_spec, _b_spec, _w_spec, _b_spec, _row_spec],
    out_specs=[_row_spec, _row_spec, _row_spec],
    out_shape=[jax.ShapeDtypeStruct((_N, _D), _f32)] * 3,
)


def kernel(x, edge_index, W_self1, W_neigh1, b1, W_self2, W_neigh2, b2,
           W_mu, b_mu, W_var, b_var, eps):
    src = edge_index[0].astype(jnp.int32)
    dst = edge_index[1].astype(jnp.int32)
    pad = _EP - _E
    srcm = jnp.concatenate([src, jnp.zeros((pad,), jnp.int32)]).reshape(_NT, _K, _CH)
    dstm = jnp.concatenate([dst, jnp.full((pad,), _N, jnp.int32)]).reshape(_NT, _K, _CH)

    z128 = jnp.zeros((_CH, _D), _f32)
    ones128 = jnp.ones((_CH, _D), _f32)

    b1r = b1.reshape(1, _D)
    b2r = b2.reshape(1, _D)
    bmur = b_mu.reshape(1, _D)
    bvarr = b_var.reshape(1, _D)

    h0 = _log1p(x)
    deg = _deg_count(dstm, z128, ones128)
    acc1 = _agg(h0, srcm, dstm, z128)
    h1 = _dense1(h0, acc1, deg, W_self1, W_neigh1, b1r)
    acc2 = _agg(h1, srcm, dstm, z128)
    h_out, mu, var = _dense2(h1, acc2, deg, W_self2, W_neigh2, b2r,
                             W_mu, bmur, W_var, bvarr, eps)
    return (h_out, mu, var)


# speedup vs baseline: 3.3645x; 1.1173x over previous
import functools

import jax
import jax.numpy as jnp
from jax import lax
from jax.experimental import pallas as pl
from jax.experimental.pallas import tpu as pltpu
from jax.experimental.pallas import tpu_sc as plsc

_N = 10000
_E = 320000
_D = 128
_NT = 32
_CH = 128
_GB = 16
_K = _GB * (-(-_E // (_NT * _CH * _GB)))
_KG = _K // _GB
_EP = _NT * _K * _CH
_NP = 10240
_RPT = _NP // 16
_NS = _RPT // _CH
_HR = _NP // _D
_L = 16

_f32 = jnp.float32


def _agg_body(h_hbm, srcm, dstm, z128, acc_out,
              src_v, dst_v, rows0, rows1,
              acc_sh, sem_g0, sem_g1, sem_s0, sem_s1):
    c = lax.axis_index("c")
    s = lax.axis_index("s")
    t = s * 2 + c
    base = s * _RPT
    rows = (rows0, rows1)
    sem_g = (sem_g0, sem_g1)
    sem_s = (sem_s0, sem_s1)

    pltpu.sync_copy(z128, rows0)
    for i in range(_NS):
        pltpu.sync_copy(rows0, acc_sh.at[pl.ds(base + i * _CH, _CH)])
    plsc.subcore_barrier()

    @pl.loop(0, _KG)
    def _group(g):
        pltpu.sync_copy(srcm.at[t, pl.ds(g * _GB, _GB)], src_v)
        pltpu.sync_copy(dstm.at[t, pl.ds(g * _GB, _GB)], dst_v)
        gat = [None, None]
        sca = [None, None]
        gat[0] = pltpu.async_copy(h_hbm.at[src_v.at[0]], rows[0], sem_g[0])
        for j in range(_GB):
            b = j & 1
            nb = b ^ 1
            if j + 1 < _GB:
                if sca[nb] is not None:
                    sca[nb].wait()
                gat[nb] = pltpu.async_copy(
                    h_hbm.at[src_v.at[j + 1]], rows[nb], sem_g[nb])
            gat[b].wait()
            sca[b] = pltpu.async_copy(
                rows[b], acc_sh.at[dst_v.at[j]], sem_s[b], add=True)
        sca[0].wait()
        sca[1].wait()

    plsc.subcore_barrier()
    for i in range(_NS):
        pltpu.sync_copy(acc_sh.at[pl.ds(base + i * _CH, _CH)], rows0)
        pltpu.sync_copy(rows0, acc_out.at[c, pl.ds(base + i * _CH, _CH)])


def _deg_body(dstm, z128, ones128, deg_out, dst_v, ones_v, acc_sh):
    c = lax.axis_index("c")
    s = lax.axis_index("s")
    t = s * 2 + c
    base = s * _RPT

    pltpu.sync_copy(z128, ones_v)
    for i in range(_NS):
        pltpu.sync_copy(ones_v, acc_sh.at[pl.ds(base + i * _CH, _CH)])
    pltpu.sync_copy(ones128, ones_v)
    plsc.subcore_barrier()

    @pl.loop(0, _KG)
    def _group(g):
        pltpu.sync_copy(dstm.at[t, pl.ds(g * _GB, _GB)], dst_v)
        for j in range(_GB):
            pltpu.sync_copy(ones_v, acc_sh.at[dst_v.at[j]], add=True)

    plsc.subcore_barrier()
    for i in range(_NS):
        pltpu.sync_copy(acc_sh.at[pl.ds(base + i * _CH, _CH)], ones_v)
        pltpu.sync_copy(ones_v, deg_out.at[c, pl.ds(base + i * _CH, _CH)])


_sc_mesh = plsc.VectorSubcoreMesh(core_axis_name="c", subcore_axis_name="s")

_agg = pl.kernel(
    _agg_body,
    out_type=jax.ShapeDtypeStruct((2, _NP, _D), _f32),
    mesh=_sc_mesh,
    scratch_types=[
        pltpu.VMEM((_GB, _CH), jnp.int32),
        pltpu.VMEM((_GB, _CH), jnp.int32),
        pltpu.VMEM((_CH, _D), _f32),
        pltpu.VMEM((_CH, _D), _f32),
        pltpu.VMEM_SHARED((_NP, _D), _f32),
        pltpu.SemaphoreType.DMA,
        pltpu.SemaphoreType.DMA,
        pltpu.SemaphoreType.DMA,
        pltpu.SemaphoreType.DMA,
    ],
)

_deg_count = pl.kernel(
    _deg_body,
    out_type=jax.ShapeDtypeStruct((2, _NP, _D), _f32),
    mesh=_sc_mesh,
    scratch_types=[
        pltpu.VMEM((_GB, _CH), jnp.int32),
        pltpu.VMEM((_CH, _D), _f32),
        pltpu.VMEM_SHARED((_NP, _D), _f32),
    ],
)


_BN = 2048
_BH = _BN // _D


def _log1p_body(x_ref, o_ref):
    o_ref[...] = jnp.log(x_ref[...] + 1.0)


def _neigh_mean(acc_ref, deg_ref):
    a = acc_ref[0] + acc_ref[1]
    deg = deg_ref[0, :, 0:1] + deg_ref[1, :, 0:1]
    return a / jnp.maximum(deg, 1.0)


def _dense1_body(h_ref, acc_ref, deg_ref, ws_ref, wn_ref, b_ref, o_ref):
    hn = _neigh_mean(acc_ref, deg_ref)
    z = (jnp.dot(h_ref[...], ws_ref[...], preferred_element_type=_f32)
         + jnp.dot(hn, wn_ref[...], preferred_element_type=_f32)
         + b_ref[...])
    z = jnp.maximum(z, 0.0)
    nrm = jnp.sqrt(jnp.sum(z * z, axis=1, keepdims=True))
    o_ref[...] = z / jnp.maximum(nrm, 1e-12)


def _dense2_body(h_ref, acc_ref, deg_ref, ws_ref, wn_ref, b_ref,
                 wmu_ref, bmu_ref, wvar_ref, bvar_ref, eps_ref,
                 out_ref, mu_ref, var_ref):
    hn = _neigh_mean(acc_ref, deg_ref)
    h2 = (jnp.dot(h_ref[...], ws_ref[...], preferred_element_type=_f32)
          + jnp.dot(hn, wn_ref[...], preferred_element_type=_f32)
          + b_ref[...])
    mu = jnp.dot(h2, wmu_ref[...], preferred_element_type=_f32) + bmu_ref[...]
    var = jnp.dot(h2, wvar_ref[...], preferred_element_type=_f32) + bvar_ref[...]
    mu_ref[...] = mu
    var_ref[...] = var
    out_ref[...] = mu + jnp.sqrt(jnp.exp(var) + 1e-8) * eps_ref[...]


_row_spec = pl.BlockSpec((_BN, _D), lambda i: (i, 0))
_acc_spec = pl.BlockSpec((2, _BN, _D), lambda i: (0, i, 0))
_deg_spec = pl.BlockSpec((2, _BN, _D), lambda i: (0, i, 0))
_w_spec = pl.BlockSpec((_D, _D), lambda i: (0, 0))
_b_spec = pl.BlockSpec((1, _D), lambda i: (0, 0))
_GRID = (-(-_N // _BN),)

_log1p = pl.pallas_call(
    _log1p_body,
    grid=_GRID,
    in_specs=[_row_spec],
    out_specs=_row_spec,
    out_shape=jax.ShapeDtypeStruct((_N, _D), _f32),
)

_dense1 = pl.pallas_call(
    _dense1_body,
    grid=_GRID,
    in_specs=[_row_spec, _acc_spec, _deg_spec, _w_spec, _w_spec, _b_spec],
    out_specs=_row_spec,
    out_shape=jax.ShapeDtypeStruct((_N, _D), _f32),
)

_dense2 = pl.pallas_call(
    _dense2_body,
    grid=_GRID,
    in_specs=[_row_spec, _acc_spec, _deg_spec, _w_spec, _w_spec, _b_spec,
              _w_spec, _b_spec, _w_spec, _b_spec, _row_spec],
    out_specs=[_row_spec, _row_spec, _row_spec],
    out_shape=[jax.ShapeDtypeStruct((_N, _D), _f32)] * 3,
)


def kernel(x, edge_index, W_self1, W_neigh1, b1, W_self2, W_neigh2, b2,
           W_mu, b_mu, W_var, b_var, eps):
    src = edge_index[0].astype(jnp.int32)
    dst = edge_index[1].astype(jnp.int32)
    pad = _EP - _E
    srcm = jnp.concatenate([src, jnp.zeros((pad,), jnp.int32)]).reshape(_NT, _K, _CH)
    dstm = jnp.concatenate([dst, jnp.full((pad,), _N, jnp.int32)]).reshape(_NT, _K, _CH)

    z128 = jnp.zeros((_CH, _D), _f32)
    ones128 = jnp.ones((_CH, _D), _f32)

    b1r = b1.reshape(1, _D)
    b2r = b2.reshape(1, _D)
    bmur = b_mu.reshape(1, _D)
    bvarr = b_var.reshape(1, _D)

    h0 = _log1p(x)
    deg = _deg_count(dstm, z128, ones128)
    acc1 = _agg(h0, srcm, dstm, z128)
    h1 = _dense1(h0, acc1, deg, W_self1, W_neigh1, b1r)
    acc2 = _agg(h1, srcm, dstm, z128)
    h_out, mu, var = _dense2(h1, acc2, deg, W_self2, W_neigh2, b2r,
                             W_mu, bmur, W_var, bvarr, eps)
    return (h_out, mu, var)


# speedup vs baseline: 3.3987x; 1.0101x over previous
import functools

import jax
import jax.numpy as jnp
from jax import lax
from jax.experimental import pallas as pl
from jax.experimental.pallas import tpu as pltpu
from jax.experimental.pallas import tpu_sc as plsc

_N = 10000
_E = 320000
_D = 128
_NT = 32
_CH = 128
_GB = 8
_K = _GB * (-(-_E // (_NT * _CH * _GB)))
_KG = _K // _GB
_EP = _NT * _K * _CH
_NC = _NT * _K // 16
_KF = 120
_KS = _NC - _KF
_NP = 10240
_RPT = _NP // 16
_NS = _RPT // _CH
_HR = _NP // _D
_L = 16

_f32 = jnp.float32


def _agg_body(h_hbm, srcm, dstm, z128, acc_out,
              src_v, dst_v, rows0, rows1,
              acc_sh, sem_g0, sem_g1, sem_s0, sem_s1):
    c = lax.axis_index("c")
    s = lax.axis_index("s")
    base = s * _RPT
    rows = (rows0, rows1)
    sem_g = (sem_g0, sem_g1)
    sem_s = (sem_s0, sem_s1)
    ngroups = jnp.where(c == 0, _KF // _GB, _KS // _GB)
    start = jnp.where(c == 0, s * _KF, 16 * _KF + s * _KS)

    pltpu.sync_copy(z128, rows0)
    for i in range(_NS):
        pltpu.sync_copy(rows0, acc_sh.at[pl.ds(base + i * _CH, _CH)])
    plsc.subcore_barrier()

    @pl.loop(0, ngroups)
    def _group(g):
        pltpu.sync_copy(srcm.at[pl.ds(start + g * _GB, _GB)], src_v)
        pltpu.sync_copy(dstm.at[pl.ds(start + g * _GB, _GB)], dst_v)
        gat = [None, None]
        sca = [None, None]
        gat[0] = pltpu.async_copy(h_hbm.at[src_v.at[0]], rows[0], sem_g[0])
        for j in range(_GB):
            b = j & 1
            nb = b ^ 1
            if j + 1 < _GB:
                if sca[nb] is not None:
                    sca[nb].wait()
                gat[nb] = pltpu.async_copy(
                    h_hbm.at[src_v.at[j + 1]], rows[nb], sem_g[nb])
            gat[b].wait()
            sca[b] = pltpu.async_copy(
                rows[b], acc_sh.at[dst_v.at[j]], sem_s[b], add=True)
        sca[0].wait()
        sca[1].wait()

    plsc.subcore_barrier()
    for i in range(_NS):
        pltpu.sync_copy(acc_sh.at[pl.ds(base + i * _CH, _CH)], rows0)
        pltpu.sync_copy(rows0, acc_out.at[c, pl.ds(base + i * _CH, _CH)])


def _deg_body(dstm, z128, ones128, deg_out, dst_v, ones_v, acc_sh):
    c = lax.axis_index("c")
    s = lax.axis_index("s")
    t = s * 2 + c
    base = s * _RPT

    pltpu.sync_copy(z128, ones_v)
    for i in range(_NS):
        pltpu.sync_copy(ones_v, acc_sh.at[pl.ds(base + i * _CH, _CH)])
    pltpu.sync_copy(ones128, ones_v)
    plsc.subcore_barrier()

    @pl.loop(0, _KG)
    def _group(g):
        pltpu.sync_copy(dstm.at[pl.ds((t * _KG + g) * _GB, _GB)], dst_v)
        for j in range(_GB):
            pltpu.sync_copy(ones_v, acc_sh.at[dst_v.at[j]], add=True)

    plsc.subcore_barrier()
    for i in range(_NS):
        pltpu.sync_copy(acc_sh.at[pl.ds(base + i * _CH, _CH)], ones_v)
        pltpu.sync_copy(ones_v, deg_out.at[c, pl.ds(base + i * _CH, _CH)])


_sc_mesh = plsc.VectorSubcoreMesh(core_axis_name="c", subcore_axis_name="s")

_agg = pl.kernel(
    _agg_body,
    out_type=jax.ShapeDtypeStruct((2, _NP, _D), _f32),
    mesh=_sc_mesh,
    scratch_types=[
        pltpu.VMEM((_GB, _CH), jnp.int32),
        pltpu.VMEM((_GB, _CH), jnp.int32),
        pltpu.VMEM((_CH, _D), _f32),
        pltpu.VMEM((_CH, _D), _f32),
        pltpu.VMEM_SHARED((_NP, _D), _f32),
        pltpu.SemaphoreType.DMA,
        pltpu.SemaphoreType.DMA,
        pltpu.SemaphoreType.DMA,
        pltpu.SemaphoreType.DMA,
    ],
)

_deg_count = pl.kernel(
    _deg_body,
    out_type=jax.ShapeDtypeStruct((2, _NP, _D), _f32),
    mesh=_sc_mesh,
    scratch_types=[
        pltpu.VMEM((_GB, _CH), jnp.int32),
        pltpu.VMEM((_CH, _D), _f32),
        pltpu.VMEM_SHARED((_NP, _D), _f32),
    ],
)


_BN = 2048
_BH = _BN // _D


def _log1p_body(x_ref, o_ref):
    o_ref[...] = jnp.log(x_ref[...] + 1.0)


def _neigh_mean(acc_ref, deg_ref):
    a = acc_ref[0] + acc_ref[1]
    deg = deg_ref[0, :, 0:1] + deg_ref[1, :, 0:1]
    return a / jnp.maximum(deg, 1.0)


def _dense1_body(h_ref, acc_ref, deg_ref, ws_ref, wn_ref, b_ref, o_ref):
    hn = _neigh_mean(acc_ref, deg_ref)
    z = (jnp.dot(h_ref[...], ws_ref[...], preferred_element_type=_f32)
         + jnp.dot(hn, wn_ref[...], preferred_element_type=_f32)
         + b_ref[...])
    z = jnp.maximum(z, 0.0)
    nrm = jnp.sqrt(jnp.sum(z * z, axis=1, keepdims=True))
    o_ref[...] = z / jnp.maximum(nrm, 1e-12)


def _dense2_body(h_ref, acc_ref, deg_ref, ws_ref, wn_ref, b_ref,
                 wmu_ref, bmu_ref, wvar_ref, bvar_ref, eps_ref,
                 out_ref, mu_ref, var_ref):
    hn = _neigh_mean(acc_ref, deg_ref)
    h2 = (jnp.dot(h_ref[...], ws_ref[...], preferred_element_type=_f32)
          + jnp.dot(hn, wn_ref[...], preferred_element_type=_f32)
          + b_ref[...])
    mu = jnp.dot(h2, wmu_ref[...], preferred_element_type=_f32) + bmu_ref[...]
    var = jnp.dot(h2, wvar_ref[...], preferred_element_type=_f32) + bvar_ref[...]
    mu_ref[...] = mu
    var_ref[...] = var
    out_ref[...] = mu + jnp.sqrt(jnp.exp(var) + 1e-8) * eps_ref[...]


_row_spec = pl.BlockSpec((_BN, _D), lambda i: (i, 0))
_acc_spec = pl.BlockSpec((2, _BN, _D), lambda i: (0, i, 0))
_deg_spec = pl.BlockSpec((2, _BN, _D), lambda i: (0, i, 0))
_w_spec = pl.BlockSpec((_D, _D), lambda i: (0, 0))
_b_spec = pl.BlockSpec((1, _D), lambda i: (0, 0))
_GRID = (-(-_N // _BN),)

_log1p = pl.pallas_call(
    _log1p_body,
    grid=_GRID,
    in_specs=[_row_spec],
    out_specs=_row_spec,
    out_shape=jax.ShapeDtypeStruct((_N, _D), _f32),
)

_dense1 = pl.pallas_call(
    _dense1_body,
    grid=_GRID,
    in_specs=[_row_spec, _acc_spec, _deg_spec, _w_spec, _w_spec, _b_spec],
    out_specs=_row_spec,
    out_shape=jax.ShapeDtypeStruct((_N, _D), _f32),
)

_dense2 = pl.pallas_call(
    _dense2_body,
    grid=_GRID,
    in_specs=[_row_spec, _acc_spec, _deg_spec, _w_spec, _w_spec, _b_spec,
              _w_spec, _b_spec, _w_spec, _b_spec, _row_spec],
    out_specs=[_row_spec, _row_spec, _row_spec],
    out_shape=[jax.ShapeDtypeStruct((_N, _D), _f32)] * 3,
)


def kernel(x, edge_index, W_self1, W_neigh1, b1, W_self2, W_neigh2, b2,
           W_mu, b_mu, W_var, b_var, eps):
    src = edge_index[0].astype(jnp.int32)
    dst = edge_index[1].astype(jnp.int32)
    pad = _EP - _E
    srcm = jnp.concatenate([src, jnp.zeros((pad,), jnp.int32)]).reshape(_NT * _K, _CH)
    dstm = jnp.concatenate([dst, jnp.full((pad,), _N, jnp.int32)]).reshape(_NT * _K, _CH)

    z128 = jnp.zeros((_CH, _D), _f32)
    ones128 = jnp.ones((_CH, _D), _f32)

    b1r = b1.reshape(1, _D)
    b2r = b2.reshape(1, _D)
    bmur = b_mu.reshape(1, _D)
    bvarr = b_var.reshape(1, _D)

    h0 = _log1p(x)
    deg = _deg_count(dstm, z128, ones128)
    acc1 = _agg(h0, srcm, dstm, z128)
    h1 = _dense1(h0, acc1, deg, W_self1, W_neigh1, b1r)
    acc2 = _agg(h1, srcm, dstm, z128)
    h_out, mu, var = _dense2(h1, acc2, deg, W_self2, W_neigh2, b2r,
                             W_mu, bmur, W_var, bvarr, eps)
    return (h_out, mu, var)
